# baseline (speedup 1.0000x reference)
import functools

import jax
import jax.numpy as jnp
from jax import lax
from jax.experimental import pallas as pl
from jax.experimental.pallas import tpu as pltpu
from jax.experimental.pallas import tpu_sc as plsc

EMBED_D = 128
TABLE_ROWS = 20
N_ROWS = 100000
NUM_CORES = 2
NUM_SUBCORES = 16
NUM_WORKERS = NUM_CORES * NUM_SUBCORES
CHUNK = 400
HALF = CHUNK // 2
NUM_CHUNKS = N_ROWS // CHUNK
MAX_ITERS = -(-NUM_CHUNKS // NUM_WORKERS)

_mesh = plsc.VectorSubcoreMesh(
    core_axis_name="c", subcore_axis_name="s",
    num_cores=NUM_CORES, num_subcores=NUM_SUBCORES)


@functools.partial(
    pl.kernel,
    mesh=_mesh,
    out_type=jax.ShapeDtypeStruct((N_ROWS, EMBED_D), jnp.float32),
    scratch_types=(
        [pltpu.VMEM_SHARED((TABLE_ROWS, EMBED_D), jnp.float32)]
        + [pltpu.VMEM((HALF, EMBED_D), jnp.float32) for _ in range(4)]
        + [pltpu.VMEM((HALF,), jnp.int32) for _ in range(2 * MAX_ITERS)]
        + [pltpu.SemaphoreType.DMA,
           pltpu.SemaphoreType.DMA,
           pltpu.SemaphoreType.DMA,
           pltpu.SemaphoreType.DMA]
    ),
)
def _embed_sc(idx_hbm, table_hbm, out_hbm, *scratch):
    table_v = scratch[0]
    rows = ((scratch[1], scratch[2]), (scratch[3], scratch[4]))
    idx_v = scratch[5:5 + 2 * MAX_ITERS]
    sem_g, sem_s0, sem_s1, sem_i = scratch[5 + 2 * MAX_ITERS:]
    sem_s = (sem_s0, sem_s1)
    wid = lax.axis_index("s") * NUM_CORES + lax.axis_index("c")

    def chunk_id(k):
        return wid + k * NUM_WORKERS

    def idx_slice(k, h):
        return idx_hbm.at[pl.ds(chunk_id(k) * CHUNK + h * HALF, HALF)]

    def out_half(k, h):
        return out_hbm.at[pl.ds(chunk_id(k) * CHUNK + h * HALF, HALF)]

    @pl.when(lax.axis_index("s") == 0)
    def _():
        pltpu.sync_copy(table_hbm, table_v)

    plsc.subcore_barrier()

    for k in range(MAX_ITERS):

        @pl.when(chunk_id(k) < NUM_CHUNKS)
        def _():
            for h in range(2):
                pltpu.async_copy(idx_slice(k, h), idx_v[2 * k + h], sem_i)

    for k in range(MAX_ITERS):

        @pl.when(chunk_id(k) < NUM_CHUNKS)
        def _():
            for h in range(2):
                pltpu.make_async_copy(idx_slice(k, h), idx_v[2 * k + h],
                                      sem_i).wait()

    for k in range(MAX_ITERS):
        buf = k % 2

        @pl.when(chunk_id(k) < NUM_CHUNKS)
        def _():
            if k >= 2:
                for h in range(2):
                    pltpu.make_async_copy(rows[buf][h], out_half(k - 2, h),
                                          sem_s[buf]).wait()
            for h in range(2):
                pltpu.async_copy(table_v.at[idx_v[2 * k + h]], rows[buf][h],
                                 sem_g)
            for h in range(2):
                pltpu.make_async_copy(table_v.at[idx_v[2 * k + h]],
                                      rows[buf][h], sem_g).wait()
            for h in range(2):
                pltpu.async_copy(rows[buf][h], out_half(k, h), sem_s[buf])

    for k in range(max(MAX_ITERS - 2, 0), MAX_ITERS):
        buf = k % 2

        @pl.when(chunk_id(k) < NUM_CHUNKS)
        def _():
            for h in range(2):
                pltpu.make_async_copy(rows[buf][h], out_half(k, h),
                                      sem_s[buf]).wait()


def kernel(atom_type_index, embedding_table):
    idx = atom_type_index.astype(jnp.int32)
    return _embed_sc(idx, embedding_table)
